# baseline (speedup 1.0000x reference)
import jax
import jax.numpy as jnp
from jax import lax
from jax.experimental import pallas as pl
from jax.experimental.pallas import tpu as pltpu

D_INT = 32
D_MODEL = 1024
N_TOK = 4096
O_SMALL = 16
O_TILE = 128


def _ray_body(Vd_ref, Rw_ref, W0_ref, Rb_ref, b0_ref, W_ref, b_ref):
    rows = Rw_ref[...].reshape(O_SMALL * D_MODEL, D_INT)
    V128 = Vd_ref[...]

    o_idx = lax.broadcasted_iota(jnp.int32, (O_SMALL, 1, O_TILE), 0)
    c_idx = lax.broadcasted_iota(jnp.int32, (O_SMALL, 1, O_TILE), 2)
    sel = (o_idx == c_idx).astype(jnp.float32)

    Wdup = jax.lax.dot_general(rows, V128, (((1,), (0,)), ((), ())))
    D3 = Wdup.reshape(O_SMALL, D_MODEL, O_TILE)
    W_ref[...] = jnp.sum(D3 * sel, axis=2) + W0_ref[...]

    bdup = jax.lax.dot_general(Rb_ref[...], V128, (((1,), (0,)), ((), ())))
    sel2 = (lax.broadcasted_iota(jnp.int32, (O_SMALL, O_TILE), 0)
            == lax.broadcasted_iota(jnp.int32, (O_SMALL, O_TILE), 1)
            ).astype(jnp.float32)
    b_ref[...] = jnp.sum(bdup * sel2, axis=1, keepdims=True) + b0_ref[...]


def _mm_body(x_ref, W_ref, b_ref, out_ref):
    acc = jax.lax.dot_general(x_ref[...], W_ref[...], (((1,), (1,)), ((), ())))
    ones = jnp.ones((N_TOK, 1), jnp.float32)
    bias = jax.lax.dot_general(ones, b_ref[...], (((1,), (1,)), ((), ())))
    out_ref[...] = acc + bias


def kernel(x, V, W0, b0, R_w, R_b):
    V128 = jnp.tile(V, (1, O_TILE))
    b0c = b0.reshape(D_MODEL, 1)

    W, bc = pl.pallas_call(
        _ray_body,
        grid=(D_MODEL // O_SMALL,),
        in_specs=[
            pl.BlockSpec((D_INT, O_TILE), lambda r: (0, 0)),
            pl.BlockSpec((O_SMALL, D_MODEL, D_INT), lambda r: (r, 0, 0)),
            pl.BlockSpec((O_SMALL, D_MODEL), lambda r: (r, 0)),
            pl.BlockSpec((O_SMALL, D_INT), lambda r: (r, 0)),
            pl.BlockSpec((O_SMALL, 1), lambda r: (r, 0)),
        ],
        out_specs=[
            pl.BlockSpec((O_SMALL, D_MODEL), lambda r: (r, 0)),
            pl.BlockSpec((O_SMALL, 1), lambda r: (r, 0)),
        ],
        out_shape=[
            jax.ShapeDtypeStruct((D_MODEL, D_MODEL), jnp.float32),
            jax.ShapeDtypeStruct((D_MODEL, 1), jnp.float32),
        ],
        compiler_params=pltpu.CompilerParams(
            dimension_semantics=("arbitrary",),
        ),
    )(V128, R_w, W0, R_b, b0c)

    return pl.pallas_call(
        _mm_body,
        grid=(D_MODEL // O_TILE,),
        in_specs=[
            pl.BlockSpec((N_TOK, D_MODEL), lambda o: (0, 0)),
            pl.BlockSpec((O_TILE, D_MODEL), lambda o: (o, 0)),
            pl.BlockSpec((O_TILE, 1), lambda o: (o, 0)),
        ],
        out_specs=pl.BlockSpec((N_TOK, O_TILE), lambda o: (0, o)),
        out_shape=jax.ShapeDtypeStruct((N_TOK, D_MODEL), jnp.float32),
        compiler_params=pltpu.CompilerParams(
            dimension_semantics=("arbitrary",),
        ),
    )(x, W, bc)
